# Initial kernel scaffold; baseline (speedup 1.0000x reference)
import jax
import jax.numpy as jnp
from jax.experimental import pallas as pl


def kernel(edge_types, radial, center_table, neighbor_table, W, b):
    raise NotImplementedError("write your pallas kernel here")


# speedup vs baseline: 4.7497x; 4.7497x over previous
import jax
import jax.numpy as jnp
from jax.experimental import pallas as pl
from jax.experimental.pallas import tpu as pltpu

_NT = 64
_NB = 16
_EMB = 32
_BE = 16_000


def _body(t0_ref, t1_ref, radial_ref, ct_ref, nt_ref, W_ref, b_ref, out_ref):
    be = t0_ref.shape[-1]
    radial = radial_ref[...]
    basis = jnp.dot(radial, W_ref[...], preferred_element_type=jnp.float32) + b_ref[...]
    t0 = t0_ref[0, 0, :]
    t1 = t1_ref[0, 0, :]
    iota = jax.lax.broadcasted_iota(jnp.int32, (be, _NT), 1)
    oh0 = (t0[:, None] == iota).astype(jnp.float32)
    oh1 = (t1[:, None] == iota).astype(jnp.float32)
    ce = jnp.dot(oh0, ct_ref[...], preferred_element_type=jnp.float32)
    ne = jnp.dot(oh1, nt_ref[...], preferred_element_type=jnp.float32)
    te = jnp.concatenate([ce, ne], axis=1)
    out_ref[...] = te * basis


def kernel(edge_types, radial, center_table, neighbor_table, W, b):
    E = radial.shape[0]
    assert E % _BE == 0
    nblk = E // _BE
    t0 = edge_types[0].astype(jnp.int32).reshape(nblk, 1, _BE)
    t1 = edge_types[1].astype(jnp.int32).reshape(nblk, 1, _BE)
    b2 = b.reshape(1, _EMB)
    return pl.pallas_call(
        _body,
        grid=(nblk,),
        in_specs=[
            pl.BlockSpec((1, 1, _BE), lambda i: (i, 0, 0)),
            pl.BlockSpec((1, 1, _BE), lambda i: (i, 0, 0)),
            pl.BlockSpec((_BE, _NB), lambda i: (i, 0)),
            pl.BlockSpec((_NT, _NB), lambda i: (0, 0)),
            pl.BlockSpec((_NT, _NB), lambda i: (0, 0)),
            pl.BlockSpec((_NB, _EMB), lambda i: (0, 0)),
            pl.BlockSpec((1, _EMB), lambda i: (0, 0)),
        ],
        out_specs=pl.BlockSpec((_BE, _EMB), lambda i: (i, 0)),
        out_shape=jax.ShapeDtypeStruct((E, _EMB), jnp.float32),
    )(t0, t1, radial, center_table, neighbor_table, W, b2)
